# baseline (speedup 1.0000x reference)
import functools

import jax
import jax.numpy as jnp
from jax import lax
from jax.experimental import pallas as pl
from jax.experimental.pallas import tpu as pltpu
from jax.experimental.pallas import tpu_sc as plsc

N_NODES = 10000
HIDDEN = 128
PRED_CH = 64
N_EDGES = 320000

NC = 2
NS = 16
N_HALF = 5056
ACC_BASE = N_NODES
DUMP_ROW = ACC_BASE + N_HALF
REGION_ROWS = DUMP_ROW + 8
CHUNK = 48
G_CH = 8
N_GROUPS = 53
C_PER_T = G_CH * N_GROUPS
E_PAD = NS * C_PER_T * CHUNK
X_ROWS_A = 624
Z_ROWS = 312
DST_PAD = REGION_ROWS

_sc_mesh = plsc.VectorSubcoreMesh(core_axis_name="c", subcore_axis_name="s")


@functools.partial(
    pl.kernel,
    out_type=jax.ShapeDtypeStruct((NC * N_HALF, HIDDEN), jnp.float32),
    mesh=_sc_mesh,
    scratch_types=[
        pltpu.VMEM((G_CH, CHUNK), jnp.int32),
        pltpu.VMEM((G_CH, CHUNK), jnp.int32),
        pltpu.VMEM((CHUNK, HIDDEN), jnp.float32),
        pltpu.VMEM_SHARED((REGION_ROWS, HIDDEN), jnp.float32),
        pltpu.SemaphoreType.DMA,
    ],
)
def _sc_segment_sum(x_hbm, src_hbm, dst_hbm, zeros_hbm, out_hbm,
                    src_v, dst_v, rows0, region, sem0):
    cid = lax.axis_index("c")
    sid = lax.axis_index("s")
    base = cid * N_HALF
    pltpu.sync_copy(x_hbm.at[pl.ds(sid * X_ROWS_A, X_ROWS_A)],
                    region.at[pl.ds(sid * X_ROWS_A, X_ROWS_A)])
    pltpu.sync_copy(zeros_hbm.at[pl.ds(0, Z_ROWS)],
                    region.at[pl.ds(ACC_BASE + sid * Z_ROWS, Z_ROWS)])

    @pl.when(sid == NS - 1)
    def _():
        xt = NS * X_ROWS_A
        pltpu.sync_copy(x_hbm.at[pl.ds(xt, N_NODES - xt)],
                        region.at[pl.ds(xt, N_NODES - xt)])
        zt = ACC_BASE + NS * Z_ROWS
        pltpu.sync_copy(zeros_hbm.at[pl.ds(0, REGION_ROWS - zt)],
                        region.at[pl.ds(zt, REGION_ROWS - zt)])

    plsc.subcore_barrier()

    def group_body(g, carry):
        pltpu.sync_copy(src_hbm.at[sid, pl.ds(g * G_CH, G_CH)], src_v)
        pltpu.sync_copy(dst_hbm.at[sid, pl.ds(g * G_CH, G_CH)], dst_v)

        def body(jj, c):
            d = pltpu.async_copy(region.at[src_v.at[jj]], rows0, sem0)
            for k in range(CHUNK // 16):
                v = dst_v[jj, pl.ds(k * 16, 16)]
                local = v - base
                inb = (local >= 0) & (local < N_HALF)
                dst_v[jj, pl.ds(k * 16, 16)] = jnp.where(
                    inb, local + ACC_BASE, DUMP_ROW)
            d.wait()
            pltpu.sync_copy(rows0, region.at[dst_v.at[jj]], add=True)
            return c

        lax.fori_loop(0, G_CH, body, 0)
        return carry

    lax.fori_loop(0, N_GROUPS, group_body, 0)
    plsc.subcore_barrier()
    pltpu.sync_copy(region.at[pl.ds(ACC_BASE + sid * Z_ROWS, Z_ROWS)],
                    out_hbm.at[pl.ds(base + sid * Z_ROWS, Z_ROWS)])

    @pl.when(sid == NS - 1)
    def _():
        zt = NS * Z_ROWS
        pltpu.sync_copy(region.at[pl.ds(ACC_BASE + zt, N_HALF - zt)],
                        out_hbm.at[pl.ds(base + zt, N_HALF - zt)])


BLK = 2000


def _tc_body(p_ref, u_ref, Wh_ref, Wu_ref, b_ref, Wp_ref, bp_ref, xn_ref, y_ref):
    agg = p_ref[...]
    h = jnp.dot(agg, Wh_ref[...], preferred_element_type=jnp.float32)
    h = h + jnp.dot(u_ref[...], Wu_ref[...], preferred_element_type=jnp.float32)
    h = h + b_ref[...]
    xn = jnp.maximum(h, 0.0)
    xn_ref[...] = xn
    y_ref[...] = jnp.dot(xn, Wp_ref[...], preferred_element_type=jnp.float32) + bp_ref[...]


_tc_update = pl.pallas_call(
    _tc_body,
    grid=(N_NODES // BLK,),
    in_specs=[
        pl.BlockSpec((BLK, HIDDEN), lambda i: (i, 0)),
        pl.BlockSpec((BLK, HIDDEN), lambda i: (i, 0)),
        pl.BlockSpec((HIDDEN, HIDDEN), lambda i: (0, 0)),
        pl.BlockSpec((HIDDEN, HIDDEN), lambda i: (0, 0)),
        pl.BlockSpec((1, HIDDEN), lambda i: (0, 0)),
        pl.BlockSpec((HIDDEN, PRED_CH), lambda i: (0, 0)),
        pl.BlockSpec((1, PRED_CH), lambda i: (0, 0)),
    ],
    out_specs=[
        pl.BlockSpec((BLK, HIDDEN), lambda i: (i, 0)),
        pl.BlockSpec((BLK, PRED_CH), lambda i: (i, 0)),
    ],
    out_shape=[
        jax.ShapeDtypeStruct((N_NODES, HIDDEN), jnp.float32),
        jax.ShapeDtypeStruct((N_NODES, PRED_CH), jnp.float32),
    ],
)


def kernel(x, u, edge_index, W_h, W_u, b, W_p, b_p):
    src = edge_index[0].astype(jnp.int32)
    dst = edge_index[1].astype(jnp.int32)
    pad = E_PAD - N_EDGES
    src = jnp.concatenate([src, jnp.zeros((pad,), jnp.int32)])
    dst = jnp.concatenate([dst, jnp.full((pad,), DST_PAD, jnp.int32)])
    src3 = src.reshape(NS, C_PER_T, CHUNK)
    dst3 = dst.reshape(NS, C_PER_T, CHUNK)
    zeros_blk = jnp.zeros((Z_ROWS, HIDDEN), jnp.float32)

    agg = _sc_segment_sum(x, src3, dst3, zeros_blk)

    x_next, y = _tc_update(agg, u, W_h, W_u, b.reshape(1, HIDDEN),
                           W_p, b_p.reshape(1, PRED_CH))
    return (x_next, y)


# speedup vs baseline: 1.2265x; 1.2265x over previous
import functools

import jax
import jax.numpy as jnp
from jax import lax
from jax.experimental import pallas as pl
from jax.experimental.pallas import tpu as pltpu
from jax.experimental.pallas import tpu_sc as plsc

N_NODES = 10000
HIDDEN = 128
PRED_CH = 64
N_EDGES = 320000

NC = 2
NS = 16
N_HALF = 5056
ACC_BASE = N_NODES
DUMP_ROW = ACC_BASE + N_HALF
REGION_ROWS = DUMP_ROW + 8
CHUNK = 32
G_CH = 8
N_GROUPS = 79
C_PER_T = G_CH * N_GROUPS
E_PAD = NS * C_PER_T * CHUNK
X_ROWS_A = 624
Z_ROWS = 312
DST_PAD = REGION_ROWS

_sc_mesh = plsc.VectorSubcoreMesh(core_axis_name="c", subcore_axis_name="s")


@functools.partial(
    pl.kernel,
    out_type=jax.ShapeDtypeStruct((NC * N_HALF, HIDDEN), jnp.float32),
    mesh=_sc_mesh,
    scratch_types=[
        pltpu.VMEM((G_CH, CHUNK), jnp.int32),
        pltpu.VMEM((G_CH, CHUNK), jnp.int32),
        pltpu.VMEM((CHUNK, HIDDEN), jnp.float32),
        pltpu.VMEM((CHUNK, HIDDEN), jnp.float32),
        pltpu.VMEM_SHARED((REGION_ROWS, HIDDEN), jnp.float32),
        pltpu.SemaphoreType.DMA,
        pltpu.SemaphoreType.DMA,
        pltpu.SemaphoreType.DMA,
        pltpu.SemaphoreType.DMA,
    ],
)
def _sc_segment_sum(x_hbm, src_hbm, dst_hbm, zeros_hbm, out_hbm,
                    src_v, dst_v, rows0, rows1, region,
                    semg0, semg1, sems0, sems1):
    cid = lax.axis_index("c")
    sid = lax.axis_index("s")
    base = cid * N_HALF
    pltpu.sync_copy(x_hbm.at[pl.ds(sid * X_ROWS_A, X_ROWS_A)],
                    region.at[pl.ds(sid * X_ROWS_A, X_ROWS_A)])
    pltpu.sync_copy(zeros_hbm.at[pl.ds(0, Z_ROWS)],
                    region.at[pl.ds(ACC_BASE + sid * Z_ROWS, Z_ROWS)])

    @pl.when(sid == NS - 1)
    def _():
        xt = NS * X_ROWS_A
        pltpu.sync_copy(x_hbm.at[pl.ds(xt, N_NODES - xt)],
                        region.at[pl.ds(xt, N_NODES - xt)])
        zt = ACC_BASE + NS * Z_ROWS
        pltpu.sync_copy(zeros_hbm.at[pl.ds(0, REGION_ROWS - zt)],
                        region.at[pl.ds(zt, REGION_ROWS - zt)])

    plsc.subcore_barrier()

    rows_b = (rows0, rows1)
    semg_b = (semg0, semg1)
    sems_b = (sems0, sems1)

    def group_body(g, carry):
        @pl.when(g > 0)
        def _():
            pltpu.make_async_copy(rows0, region.at[dst_v.at[G_CH - 2]], sems0).wait()
            pltpu.make_async_copy(rows1, region.at[dst_v.at[G_CH - 1]], sems1).wait()

        pltpu.sync_copy(src_hbm.at[sid, pl.ds(g * G_CH, G_CH)], src_v)
        pltpu.sync_copy(dst_hbm.at[sid, pl.ds(g * G_CH, G_CH)], dst_v)

        for jj in range(G_CH):
            p = jj % 2
            if jj >= 2:
                pltpu.make_async_copy(rows_b[p], region.at[dst_v.at[jj - 2]],
                                      sems_b[p]).wait()
            pltpu.async_copy(region.at[src_v.at[jj]], rows_b[p], semg_b[p])
            for k in range(CHUNK // 16):
                v = dst_v[jj, pl.ds(k * 16, 16)]
                local = v - base
                inb = (local >= 0) & (local < N_HALF)
                dst_v[jj, pl.ds(k * 16, 16)] = jnp.where(
                    inb, local + ACC_BASE, DUMP_ROW)
            pltpu.make_async_copy(region.at[src_v.at[jj]], rows_b[p],
                                  semg_b[p]).wait()
            pltpu.async_copy(rows_b[p], region.at[dst_v.at[jj]], sems_b[p],
                             add=True)
        return carry

    lax.fori_loop(0, N_GROUPS, group_body, 0)
    pltpu.make_async_copy(rows0, region.at[dst_v.at[G_CH - 2]], sems0).wait()
    pltpu.make_async_copy(rows1, region.at[dst_v.at[G_CH - 1]], sems1).wait()
    plsc.subcore_barrier()
    pltpu.sync_copy(region.at[pl.ds(ACC_BASE + sid * Z_ROWS, Z_ROWS)],
                    out_hbm.at[pl.ds(base + sid * Z_ROWS, Z_ROWS)])

    @pl.when(sid == NS - 1)
    def _():
        zt = NS * Z_ROWS
        pltpu.sync_copy(region.at[pl.ds(ACC_BASE + zt, N_HALF - zt)],
                        out_hbm.at[pl.ds(base + zt, N_HALF - zt)])


BLK = 2000


def _tc_body(p_ref, u_ref, Wh_ref, Wu_ref, b_ref, Wp_ref, bp_ref, xn_ref, y_ref):
    agg = p_ref[...]
    h = jnp.dot(agg, Wh_ref[...], preferred_element_type=jnp.float32)
    h = h + jnp.dot(u_ref[...], Wu_ref[...], preferred_element_type=jnp.float32)
    h = h + b_ref[...]
    xn = jnp.maximum(h, 0.0)
    xn_ref[...] = xn
    y_ref[...] = jnp.dot(xn, Wp_ref[...], preferred_element_type=jnp.float32) + bp_ref[...]


_tc_update = pl.pallas_call(
    _tc_body,
    grid=(N_NODES // BLK,),
    in_specs=[
        pl.BlockSpec((BLK, HIDDEN), lambda i: (i, 0)),
        pl.BlockSpec((BLK, HIDDEN), lambda i: (i, 0)),
        pl.BlockSpec((HIDDEN, HIDDEN), lambda i: (0, 0)),
        pl.BlockSpec((HIDDEN, HIDDEN), lambda i: (0, 0)),
        pl.BlockSpec((1, HIDDEN), lambda i: (0, 0)),
        pl.BlockSpec((HIDDEN, PRED_CH), lambda i: (0, 0)),
        pl.BlockSpec((1, PRED_CH), lambda i: (0, 0)),
    ],
    out_specs=[
        pl.BlockSpec((BLK, HIDDEN), lambda i: (i, 0)),
        pl.BlockSpec((BLK, PRED_CH), lambda i: (i, 0)),
    ],
    out_shape=[
        jax.ShapeDtypeStruct((N_NODES, HIDDEN), jnp.float32),
        jax.ShapeDtypeStruct((N_NODES, PRED_CH), jnp.float32),
    ],
)


def kernel(x, u, edge_index, W_h, W_u, b, W_p, b_p):
    src = edge_index[0].astype(jnp.int32)
    dst = edge_index[1].astype(jnp.int32)
    pad = E_PAD - N_EDGES
    src = jnp.concatenate([src, jnp.zeros((pad,), jnp.int32)])
    dst = jnp.concatenate([dst, jnp.full((pad,), DST_PAD, jnp.int32)])
    src3 = src.reshape(NS, C_PER_T, CHUNK)
    dst3 = dst.reshape(NS, C_PER_T, CHUNK)
    zeros_blk = jnp.zeros((Z_ROWS, HIDDEN), jnp.float32)

    agg = _sc_segment_sum(x, src3, dst3, zeros_blk)

    x_next, y = _tc_update(agg, u, W_h, W_u, b.reshape(1, HIDDEN),
                           W_p, b_p.reshape(1, PRED_CH))
    return (x_next, y)


# speedup vs baseline: 1.3355x; 1.0888x over previous
import functools

import jax
import jax.numpy as jnp
from jax import lax
from jax.experimental import pallas as pl
from jax.experimental.pallas import tpu as pltpu
from jax.experimental.pallas import tpu_sc as plsc

N_NODES = 10000
HIDDEN = 128
PRED_CH = 64
N_EDGES = 320000

NC = 2
NS = 16
NW = NC * NS
CHUNK = 128
C_PER_W = 79
E_PAD = NW * C_PER_W * CHUNK
N_ACC = 10240
ROWS_PER_TILE = N_ACC // NS

_sc_mesh = plsc.VectorSubcoreMesh(core_axis_name="c", subcore_axis_name="s")


@functools.partial(
    pl.kernel,
    out_type=jax.ShapeDtypeStruct((NC, N_ACC, HIDDEN), jnp.float32),
    mesh=_sc_mesh,
    scratch_types=[
        pltpu.VMEM((C_PER_W, CHUNK), jnp.int32),
        pltpu.VMEM((C_PER_W, CHUNK), jnp.int32),
        pltpu.VMEM((CHUNK, HIDDEN), jnp.float32),
        pltpu.VMEM_SHARED((N_ACC, HIDDEN), jnp.float32),
        pltpu.SemaphoreType.DMA,
    ],
)
def _sc_segment_sum(x_hbm, src_hbm, dst_hbm, zeros_hbm, out_hbm,
                    src_v, dst_v, rows_v, acc, sem):
    cid = lax.axis_index("c")
    sid = lax.axis_index("s")
    wid = sid * NC + cid
    pltpu.sync_copy(zeros_hbm, acc.at[pl.ds(sid * ROWS_PER_TILE, ROWS_PER_TILE)])
    pltpu.sync_copy(src_hbm.at[wid], src_v)
    pltpu.sync_copy(dst_hbm.at[wid], dst_v)
    plsc.subcore_barrier()

    def body(j, carry):
        pltpu.async_copy(x_hbm.at[src_v.at[j]], rows_v, sem).wait()
        pltpu.sync_copy(rows_v, acc.at[dst_v.at[j]], add=True)
        return carry

    lax.fori_loop(0, C_PER_W, body, 0)
    plsc.subcore_barrier()
    pltpu.sync_copy(acc.at[pl.ds(sid * ROWS_PER_TILE, ROWS_PER_TILE)],
                    out_hbm.at[cid, pl.ds(sid * ROWS_PER_TILE, ROWS_PER_TILE)])


BLK = 1000


def _tc_body(p_ref, u_ref, Wh_ref, Wu_ref, b_ref, Wp_ref, bp_ref, xn_ref, y_ref):
    agg = p_ref[0] + p_ref[1]
    h = jnp.dot(agg, Wh_ref[...], preferred_element_type=jnp.float32)
    h = h + jnp.dot(u_ref[...], Wu_ref[...], preferred_element_type=jnp.float32)
    h = h + b_ref[...]
    xn = jnp.maximum(h, 0.0)
    xn_ref[...] = xn
    y_ref[...] = jnp.dot(xn, Wp_ref[...], preferred_element_type=jnp.float32) + bp_ref[...]


_tc_update = pl.pallas_call(
    _tc_body,
    grid=(N_NODES // BLK,),
    in_specs=[
        pl.BlockSpec((NC, BLK, HIDDEN), lambda i: (0, i, 0)),
        pl.BlockSpec((BLK, HIDDEN), lambda i: (i, 0)),
        pl.BlockSpec((HIDDEN, HIDDEN), lambda i: (0, 0)),
        pl.BlockSpec((HIDDEN, HIDDEN), lambda i: (0, 0)),
        pl.BlockSpec((1, HIDDEN), lambda i: (0, 0)),
        pl.BlockSpec((HIDDEN, PRED_CH), lambda i: (0, 0)),
        pl.BlockSpec((1, PRED_CH), lambda i: (0, 0)),
    ],
    out_specs=[
        pl.BlockSpec((BLK, HIDDEN), lambda i: (i, 0)),
        pl.BlockSpec((BLK, PRED_CH), lambda i: (i, 0)),
    ],
    out_shape=[
        jax.ShapeDtypeStruct((N_NODES, HIDDEN), jnp.float32),
        jax.ShapeDtypeStruct((N_NODES, PRED_CH), jnp.float32),
    ],
)


def kernel(x, u, edge_index, W_h, W_u, b, W_p, b_p):
    src = edge_index[0].astype(jnp.int32)
    dst = edge_index[1].astype(jnp.int32)
    pad = E_PAD - N_EDGES
    src = jnp.concatenate([src, jnp.full((pad,), N_NODES, jnp.int32)])
    dst = jnp.concatenate([dst, jnp.zeros((pad,), jnp.int32)])
    src3 = src.reshape(NW, C_PER_W, CHUNK)
    dst3 = dst.reshape(NW, C_PER_W, CHUNK)
    x_pad = jnp.concatenate([x, jnp.zeros((1, HIDDEN), x.dtype)], axis=0)
    zeros_blk = jnp.zeros((ROWS_PER_TILE, HIDDEN), jnp.float32)

    partial = _sc_segment_sum(x_pad, src3, dst3, zeros_blk)

    x_next, y = _tc_update(partial, u, W_h, W_u, b.reshape(1, HIDDEN),
                           W_p, b_p.reshape(1, PRED_CH))
    return (x_next, y)
